# baseline (speedup 1.0000x reference)
import functools

import jax
import jax.numpy as jnp
from jax import lax
from jax.experimental import pallas as pl
from jax.experimental.pallas import tpu as pltpu
from jax.experimental.pallas import tpu_sc as plsc

_ATTRIBUTES_NUM = 8


@functools.lru_cache(maxsize=None)
def _make_scs_gather(seq_len, e_dims, table_rows):
    mesh = plsc.ScalarSubcoreMesh(axis_name="c")

    @functools.partial(
        pl.kernel,
        mesh=mesh,
        out_type=jax.ShapeDtypeStruct((seq_len, e_dims), jnp.float32),
        scratch_types=[
            pltpu.VMEM_SHARED((seq_len, e_dims), jnp.float32),
            pltpu.SemaphoreType.DMA,
        ],
    )
    def scs_gather(table_hbm, seq_hbm, stage, gsem):
        cid = lax.axis_index("c")

        @pl.when(cid == 0)
        def _():
            gathers = [
                pltpu.async_copy(
                    table_hbm.at[pl.ds(t // _ATTRIBUTES_NUM, 1)],
                    stage.at[pl.ds(t, 1)],
                    gsem,
                )
                for t in range(seq_len)
            ]
            for g in gathers:
                g.wait()
            pltpu.sync_copy(stage, seq_hbm)

    return scs_gather


@functools.lru_cache(maxsize=None)
def _make_tc_broadcast(batch, seq_len, e_dims, k_rep):
    nchunks = batch // k_rep

    def body(seq_ref, out_ref, scratch_ref, sem):
        seq = seq_ref[:]
        for i in range(k_rep):
            scratch_ref[i] = seq
        copies = [
            pltpu.make_async_copy(
                scratch_ref,
                out_ref.at[pl.ds(c * k_rep, k_rep)],
                sem.at[c % 2],
            )
            for c in range(nchunks)
        ]
        for cp in copies:
            cp.start()
        for cp in copies:
            cp.wait()

    return pl.pallas_call(
        body,
        in_specs=[pl.BlockSpec(memory_space=pltpu.VMEM)],
        out_specs=pl.BlockSpec(memory_space=pltpu.MemorySpace.HBM),
        out_shape=jax.ShapeDtypeStruct((batch, seq_len, e_dims),
                                       jnp.float32),
        scratch_shapes=[
            pltpu.VMEM((k_rep, seq_len, e_dims), jnp.float32),
            pltpu.SemaphoreType.DMA((2,)),
        ],
    )


def kernel(x, E_object_index):
    batch, seq_len = x.shape
    table_rows, e_dims = E_object_index.shape
    gather = _make_scs_gather(seq_len, e_dims, table_rows)
    seq = gather(E_object_index)
    broadcast = _make_tc_broadcast(batch, seq_len, e_dims, k_rep=16)
    return broadcast(seq)


# speedup vs baseline: 1.0086x; 1.0086x over previous
import functools

import jax
import jax.numpy as jnp
from jax import lax
from jax.experimental import pallas as pl
from jax.experimental.pallas import tpu as pltpu
from jax.experimental.pallas import tpu_sc as plsc

_ATTRIBUTES_NUM = 8


@functools.lru_cache(maxsize=None)
def _make_scs_gather(seq_len, e_dims, table_rows):
    mesh = plsc.ScalarSubcoreMesh(axis_name="c")
    half = ((seq_len // 2 + 7) // 8) * 8

    @functools.partial(
        pl.kernel,
        mesh=mesh,
        out_type=jax.ShapeDtypeStruct((seq_len, e_dims), jnp.float32),
        scratch_types=[
            pltpu.VMEM_SHARED((seq_len, e_dims), jnp.float32),
            pltpu.SemaphoreType.DMA,
        ],
    )
    def scs_gather(table_hbm, seq_hbm, stage, gsem):
        cid = lax.axis_index("c")

        for core, lo, ln in ((0, 0, half), (1, half, seq_len - half)):
            @pl.when(cid == core)
            def _(lo=lo, ln=ln):
                gathers = [
                    pltpu.async_copy(
                        table_hbm.at[pl.ds(t // _ATTRIBUTES_NUM, 1)],
                        stage.at[pl.ds(t, 1)],
                        gsem,
                    )
                    for t in range(lo, lo + ln)
                ]
                for g in gathers:
                    g.wait()
                pltpu.sync_copy(stage.at[pl.ds(lo, ln)],
                                seq_hbm.at[pl.ds(lo, ln)])

    return scs_gather


@functools.lru_cache(maxsize=None)
def _make_tc_broadcast(batch, seq_len, e_dims, k_rep):
    nchunks = batch // k_rep

    def body(seq_ref, out_ref, scratch_ref, sem):
        seq = seq_ref[:]
        for i in range(k_rep):
            scratch_ref[i] = seq
        copies = [
            pltpu.make_async_copy(
                scratch_ref,
                out_ref.at[pl.ds(c * k_rep, k_rep)],
                sem.at[c % 2],
            )
            for c in range(nchunks)
        ]
        for cp in copies:
            cp.start()
        for cp in copies:
            cp.wait()

    return pl.pallas_call(
        body,
        in_specs=[pl.BlockSpec(memory_space=pltpu.VMEM)],
        out_specs=pl.BlockSpec(memory_space=pltpu.MemorySpace.HBM),
        out_shape=jax.ShapeDtypeStruct((batch, seq_len, e_dims),
                                       jnp.float32),
        scratch_shapes=[
            pltpu.VMEM((k_rep, seq_len, e_dims), jnp.float32),
            pltpu.SemaphoreType.DMA((2,)),
        ],
    )


def kernel(x, E_object_index):
    batch, seq_len = x.shape
    table_rows, e_dims = E_object_index.shape
    gather = _make_scs_gather(seq_len, e_dims, table_rows)
    seq = gather(E_object_index)
    broadcast = _make_tc_broadcast(batch, seq_len, e_dims, k_rep=16)
    return broadcast(seq)
